# Initial kernel scaffold; baseline (speedup 1.0000x reference)
import jax
import jax.numpy as jnp
from jax.experimental import pallas as pl


def kernel(hv, graph_ids, W_gate, b_gate, W_g, b_g):
    raise NotImplementedError("write your pallas kernel here")


# speedup vs baseline: 4.0754x; 4.0754x over previous
import functools

import jax
import jax.numpy as jnp
from jax import lax
from jax.experimental import pallas as pl
from jax.experimental.pallas import tpu as pltpu
from jax.experimental.pallas import tpu_sc as plsc

N_NODES = 50000
D = 256
NUM_GRAPHS = 16
D_GRAPH = 2 * D

L = 16
NC = 2
NS = 16
NW = NC * NS
C = 80
NCHUNK = N_NODES // C
TPW = -(-NCHUNK // NW)
KV = D // L
ROWW = D + L
ACC_W = NUM_GRAPHS * ROWW


def _sc_body(hv_hbm, gid_hbm, wg_hbm, bg_hbm, part_hbm,
             hv_buf, gid_buf, wg_buf, bg_buf, acc):
    wid = lax.axis_index("s") * NC + lax.axis_index("c")

    zeros = jnp.zeros((L,), jnp.float32)
    for i in range(ACC_W // L):
        acc[pl.ds(L * i, L)] = zeros

    pltpu.sync_copy(wg_hbm, wg_buf)
    pltpu.sync_copy(bg_hbm, bg_buf)
    wg = [wg_buf[pl.ds(L * k, L)] for k in range(KV)]
    bgv = bg_buf[...]

    def node_body(j, _):
        row = [hv_buf[pl.ds(j * D + L * k, L)] for k in range(KV)]
        dot = row[0] * wg[0]
        for k in range(1, KV):
            dot = dot + row[k] * wg[k]
        z = jnp.sum(dot)
        zv = jnp.full((L,), z, jnp.float32) + bgv
        g = 1.0 / (1.0 + jnp.exp(-zv))
        base = gid_buf[pl.ds(j, L)][0] * ROWW
        for k in range(KV):
            plsc.addupdate(acc.at[pl.ds(base + L * k, L)], g * row[k])
        plsc.addupdate(acc.at[pl.ds(base + D, L)], g)
        return 0

    def chunk_body(t, _):
        c = wid + NW * t

        @pl.when(c < NCHUNK)
        def _():
            pltpu.sync_copy(hv_hbm.at[pl.ds(c * (C * D), C * D)], hv_buf)
            pltpu.sync_copy(gid_hbm.at[pl.ds(c * C, C)], gid_buf.at[pl.ds(0, C)])
            lax.fori_loop(0, C, node_body, 0)

        return 0

    lax.fori_loop(0, TPW, chunk_body, 0)

    pltpu.sync_copy(acc, part_hbm.at[wid])


@functools.partial(
    pl.kernel,
    out_type=jax.ShapeDtypeStruct((NW, ACC_W), jnp.float32),
    mesh=plsc.VectorSubcoreMesh(core_axis_name="c", subcore_axis_name="s"),
    compiler_params=pltpu.CompilerParams(needs_layout_passes=False),
    scratch_types=[
        pltpu.VMEM((C * D,), jnp.float32),
        pltpu.VMEM((C + L,), jnp.int32),
        pltpu.VMEM((D,), jnp.float32),
        pltpu.VMEM((L,), jnp.float32),
        pltpu.VMEM((ACC_W,), jnp.float32),
    ],
)
def _sc_segment_pass(*refs):
    _sc_body(*refs)


def _combine_body(part_ref, wg_ref, bg_ref, out_ref):
    part = part_ref[...]
    s = jnp.sum(part[:, :, :D], axis=0)
    c = jnp.sum(part[:, :, D], axis=0)
    out = jnp.dot(s, wg_ref[...], preferred_element_type=jnp.float32)
    out_ref[...] = out + c[:, None] * bg_ref[...][None, :]


def kernel(hv, graph_ids, W_gate, b_gate, W_g, b_g):
    hv_flat = hv.reshape(N_NODES * D)
    gid = graph_ids.astype(jnp.int32)
    wg_flat = W_gate.reshape(D)
    bg16 = jnp.broadcast_to(b_gate.reshape(1), (L,)).astype(jnp.float32)

    part = _sc_segment_pass(hv_flat, gid, wg_flat, bg16)
    part3 = part.reshape(NW, NUM_GRAPHS, ROWW)

    out = pl.pallas_call(
        _combine_body,
        out_shape=jax.ShapeDtypeStruct((NUM_GRAPHS, D_GRAPH), jnp.float32),
    )(part3, W_g, b_g)
    return out


# speedup vs baseline: 4.8703x; 1.1950x over previous
import functools

import jax
import jax.numpy as jnp
from jax import lax
from jax.experimental import pallas as pl
from jax.experimental.pallas import tpu as pltpu
from jax.experimental.pallas import tpu_sc as plsc

N_NODES = 50000
D = 256
NUM_GRAPHS = 16
D_GRAPH = 2 * D

L = 16
NC = 2
NS = 16
NW = NC * NS
C = 80
NCHUNK = N_NODES // C
TPW = -(-NCHUNK // NW)
KV = D // L
ROWW = D + L
ACC_W = NUM_GRAPHS * ROWW


def _sc_body(hv_hbm, gid_hbm, wg_hbm, bg_hbm, part_hbm,
             hv_a, hv_b, gid_a, gid_b, wg_buf, bg_buf, acc, sem_a, sem_b):
    wid = lax.axis_index("s") * NC + lax.axis_index("c")

    zeros = jnp.zeros((L,), jnp.float32)
    for i in range(ACC_W // L):
        acc[pl.ds(L * i, L)] = zeros

    pltpu.sync_copy(wg_hbm, wg_buf)
    pltpu.sync_copy(bg_hbm, bg_buf)
    wg = [wg_buf[pl.ds(L * k, L)] for k in range(KV)]
    bgv = bg_buf[...]

    def issue(t, hv_buf, gid_buf, sem):
        c = wid + NW * t

        @pl.when(c < NCHUNK)
        def _():
            pltpu.async_copy(hv_hbm.at[pl.ds(c * (C * D), C * D)], hv_buf, sem)
            pltpu.async_copy(gid_hbm.at[pl.ds(c * C, C)],
                             gid_buf.at[pl.ds(0, C)], sem)

    def wait(t, hv_buf, gid_buf, sem):
        c = wid + NW * t

        @pl.when(c < NCHUNK)
        def _():
            pltpu.make_async_copy(
                hv_hbm.at[pl.ds(0, C * D)], hv_buf, sem).wait()
            pltpu.make_async_copy(
                gid_hbm.at[pl.ds(0, C)], gid_buf.at[pl.ds(0, C)], sem).wait()

    def process(t, hv_buf, gid_buf):
        c = wid + NW * t

        @pl.when(c < NCHUNK)
        def _():
            def node_body(j, _):
                row = [hv_buf[pl.ds(j * D + L * k, L)] for k in range(KV)]
                p = [row[k] * wg[k] for k in range(4)]
                for k in range(4, KV):
                    p[k % 4] = p[k % 4] + row[k] * wg[k]
                dot = (p[0] + p[1]) + (p[2] + p[3])
                z = jnp.sum(dot)
                zv = jnp.full((L,), z, jnp.float32) + bgv
                g = 1.0 / (1.0 + jnp.exp(-zv))
                base = gid_buf[pl.ds(j, L)][0] * ROWW
                for k in range(KV):
                    plsc.addupdate(acc.at[pl.ds(base + L * k, L)], g * row[k])
                plsc.addupdate(acc.at[pl.ds(base + D, L)], g)
                return 0

            lax.fori_loop(0, C, node_body, 0, unroll=2)

    issue(0, hv_a, gid_a, sem_a)
    for i in range(TPW // 2):
        ta, tb = 2 * i, 2 * i + 1
        wait(ta, hv_a, gid_a, sem_a)
        issue(tb, hv_b, gid_b, sem_b)
        process(ta, hv_a, gid_a)
        wait(tb, hv_b, gid_b, sem_b)
        if tb + 1 < TPW:
            issue(tb + 1, hv_a, gid_a, sem_a)
        process(tb, hv_b, gid_b)

    pltpu.sync_copy(acc, part_hbm.at[wid])


@functools.partial(
    pl.kernel,
    out_type=jax.ShapeDtypeStruct((NW, ACC_W), jnp.float32),
    mesh=plsc.VectorSubcoreMesh(core_axis_name="c", subcore_axis_name="s"),
    compiler_params=pltpu.CompilerParams(needs_layout_passes=False),
    scratch_types=[
        pltpu.VMEM((C * D,), jnp.float32),
        pltpu.VMEM((C * D,), jnp.float32),
        pltpu.VMEM((C + L,), jnp.int32),
        pltpu.VMEM((C + L,), jnp.int32),
        pltpu.VMEM((D,), jnp.float32),
        pltpu.VMEM((L,), jnp.float32),
        pltpu.VMEM((ACC_W,), jnp.float32),
        pltpu.SemaphoreType.DMA,
        pltpu.SemaphoreType.DMA,
    ],
)
def _sc_segment_pass(*refs):
    _sc_body(*refs)


def _combine_body(part_ref, wg_ref, bg_ref, out_ref):
    part = part_ref[...]
    s = jnp.sum(part[:, :, :D], axis=0)
    c = jnp.sum(part[:, :, D], axis=0)
    out = jnp.dot(s, wg_ref[...], preferred_element_type=jnp.float32)
    out_ref[...] = out + c[:, None] * bg_ref[...][None, :]


def kernel(hv, graph_ids, W_gate, b_gate, W_g, b_g):
    hv_flat = hv.reshape(N_NODES * D)
    gid = graph_ids.astype(jnp.int32)
    wg_flat = W_gate.reshape(D)
    bg16 = jnp.broadcast_to(b_gate.reshape(1), (L,)).astype(jnp.float32)

    part = _sc_segment_pass(hv_flat, gid, wg_flat, bg16)
    part3 = part.reshape(NW, NUM_GRAPHS, ROWW)

    out = pl.pallas_call(
        _combine_body,
        out_shape=jax.ShapeDtypeStruct((NUM_GRAPHS, D_GRAPH), jnp.float32),
    )(part3, W_g, b_g)
    return out


# speedup vs baseline: 6.9015x; 1.4171x over previous
import functools

import jax
import jax.numpy as jnp
from jax import lax
from jax.experimental import pallas as pl
from jax.experimental.pallas import tpu as pltpu
from jax.experimental.pallas import tpu_sc as plsc

N_NODES = 50000
D = 256
NUM_GRAPHS = 16
D_GRAPH = 2 * D

L = 16
NC = 2
NS = 16
NW = NC * NS
C = 80
NCHUNK = N_NODES // C
TPW = -(-NCHUNK // NW)
KV = D // L
ROWW = D + L
ACC_W = NUM_GRAPHS * ROWW


def _sc_body(hv_hbm, gid_hbm, wg_hbm, bg_hbm, part_hbm,
             hv_a, hv_b, gid_a, gid_b, wg_buf, bg_buf, acc, sem_a, sem_b):
    wid = lax.axis_index("s") * NC + lax.axis_index("c")

    zeros = jnp.zeros((L,), jnp.float32)
    for i in range(ACC_W // L):
        acc[pl.ds(L * i, L)] = zeros

    pltpu.sync_copy(wg_hbm, wg_buf)
    pltpu.sync_copy(bg_hbm, bg_buf)
    wg = [wg_buf[pl.ds(L * k, L)] for k in range(KV)]
    bgv = bg_buf[...]

    def issue(t, hv_buf, gid_buf, sem):
        c = wid + NW * t

        @pl.when(c < NCHUNK)
        def _():
            pltpu.async_copy(hv_hbm.at[pl.ds(c * C, C), :], hv_buf, sem)
            pltpu.async_copy(gid_hbm.at[pl.ds(c * C, C)],
                             gid_buf.at[pl.ds(0, C)], sem)

    def wait(t, hv_buf, gid_buf, sem):
        c = wid + NW * t

        @pl.when(c < NCHUNK)
        def _():
            pltpu.make_async_copy(
                hv_hbm.at[pl.ds(0, C), :], hv_buf, sem).wait()
            pltpu.make_async_copy(
                gid_hbm.at[pl.ds(0, C)], gid_buf.at[pl.ds(0, C)], sem).wait()

    def process(t, hv_buf, gid_buf):
        c = wid + NW * t

        @pl.when(c < NCHUNK)
        def _():
            def node_body(j, _):
                row = [hv_buf[j, pl.ds(L * k, L)] for k in range(KV)]
                p = [row[k] * wg[k] for k in range(4)]
                for k in range(4, KV):
                    p[k % 4] = p[k % 4] + row[k] * wg[k]
                dot = (p[0] + p[1]) + (p[2] + p[3])
                z = jnp.sum(dot)
                zv = jnp.full((L,), z, jnp.float32) + bgv
                g = 1.0 / (1.0 + jnp.exp(-zv))
                base = gid_buf[pl.ds(j, L)][0] * ROWW
                for k in range(KV):
                    plsc.addupdate(acc.at[pl.ds(base + L * k, L)], g * row[k])
                plsc.addupdate(acc.at[pl.ds(base + D, L)], g)
                return 0

            lax.fori_loop(0, C, node_body, 0, unroll=2)

    issue(0, hv_a, gid_a, sem_a)
    for i in range(TPW // 2):
        ta, tb = 2 * i, 2 * i + 1
        wait(ta, hv_a, gid_a, sem_a)
        issue(tb, hv_b, gid_b, sem_b)
        process(ta, hv_a, gid_a)
        wait(tb, hv_b, gid_b, sem_b)
        if tb + 1 < TPW:
            issue(tb + 1, hv_a, gid_a, sem_a)
        process(tb, hv_b, gid_b)

    pltpu.sync_copy(acc, part_hbm.at[wid])


@functools.partial(
    pl.kernel,
    out_type=jax.ShapeDtypeStruct((NW, ACC_W), jnp.float32),
    mesh=plsc.VectorSubcoreMesh(core_axis_name="c", subcore_axis_name="s"),
    compiler_params=pltpu.CompilerParams(needs_layout_passes=False),
    scratch_types=[
        pltpu.VMEM((C, D), jnp.float32),
        pltpu.VMEM((C, D), jnp.float32),
        pltpu.VMEM((C + L,), jnp.int32),
        pltpu.VMEM((C + L,), jnp.int32),
        pltpu.VMEM((D,), jnp.float32),
        pltpu.VMEM((L,), jnp.float32),
        pltpu.VMEM((ACC_W,), jnp.float32),
        pltpu.SemaphoreType.DMA,
        pltpu.SemaphoreType.DMA,
    ],
)
def _sc_segment_pass(*refs):
    _sc_body(*refs)


def _combine_body(part_ref, wg_ref, bg_ref, out_ref):
    part = part_ref[...]
    s = jnp.sum(part[:, :, :D], axis=0)
    c = jnp.sum(part[:, :, D], axis=0)
    out = jnp.dot(s, wg_ref[...], preferred_element_type=jnp.float32)
    out_ref[...] = out + c[:, None] * bg_ref[...][None, :]


def kernel(hv, graph_ids, W_gate, b_gate, W_g, b_g):
    gid = graph_ids.astype(jnp.int32)
    wg_flat = W_gate.reshape(D)
    bg16 = jnp.broadcast_to(b_gate.reshape(1), (L,)).astype(jnp.float32)

    part = _sc_segment_pass(hv, gid, wg_flat, bg16)
    part3 = part.reshape(NW, NUM_GRAPHS, ROWW)

    out = pl.pallas_call(
        _combine_body,
        out_shape=jax.ShapeDtypeStruct((NUM_GRAPHS, D_GRAPH), jnp.float32),
    )(part3, W_g, b_g)
    return out


# speedup vs baseline: 8.4641x; 1.2264x over previous
import functools

import jax
import jax.numpy as jnp
from jax import lax
from jax.experimental import pallas as pl
from jax.experimental.pallas import tpu as pltpu
from jax.experimental.pallas import tpu_sc as plsc

N_NODES = 50000
D = 256
NUM_GRAPHS = 16
D_GRAPH = 2 * D

L = 16
NC = 2
NS = 16
NW = NC * NS
C = 80
NCHUNK = N_NODES // C
TPW = -(-NCHUNK // NW)
KV = D // L
ROWW = D + L
ACC_W = NUM_GRAPHS * ROWW


def _sc_body(hv_hbm, gid_hbm, wg_hbm, bg_hbm, part_hbm,
             hv_a, hv_b, gid_a, gid_b, wg_buf, bg_buf, acc, dotbuf,
             sem_a, sem_b):
    wid = lax.axis_index("s") * NC + lax.axis_index("c")

    zeros = jnp.zeros((L,), jnp.float32)
    for i in range(ACC_W // L):
        acc[pl.ds(L * i, L)] = zeros

    pltpu.sync_copy(wg_hbm, wg_buf)
    pltpu.sync_copy(bg_hbm, bg_buf)
    wg = [wg_buf[pl.ds(L * k, L)] for k in range(KV)]
    bgv = bg_buf[...]

    def issue(t, hv_buf, gid_buf, sem):
        c = wid + NW * t

        @pl.when(c < NCHUNK)
        def _():
            pltpu.async_copy(hv_hbm.at[pl.ds(c * C, C), :], hv_buf, sem)
            pltpu.async_copy(gid_hbm.at[pl.ds(c * C, C)],
                             gid_buf.at[pl.ds(0, C)], sem)

    def wait(t, hv_buf, gid_buf, sem):
        c = wid + NW * t

        @pl.when(c < NCHUNK)
        def _():
            pltpu.make_async_copy(
                hv_hbm.at[pl.ds(0, C), :], hv_buf, sem).wait()
            pltpu.make_async_copy(
                gid_hbm.at[pl.ds(0, C)], gid_buf.at[pl.ds(0, C)], sem).wait()

    iota16 = lax.iota(jnp.int32, L)
    idx0 = iota16 * L

    def process(t, hv_buf, gid_buf):
        c = wid + NW * t

        @pl.when(c < NCHUNK)
        def _():
            def dot_body(j, _):
                row = [hv_buf[j, pl.ds(L * k, L)] for k in range(KV)]
                p = [row[k] * wg[k] for k in range(4)]
                for k in range(4, KV):
                    p[k % 4] = p[k % 4] + row[k] * wg[k]
                dotbuf[pl.ds(j * L, L)] = (p[0] + p[1]) + (p[2] + p[3])
                return 0

            lax.fori_loop(0, C, dot_body, 0, unroll=2)

            def group_body(g, _):
                gbase = g * (L * L)
                z = plsc.load_gather(dotbuf, [idx0 + gbase])
                for l in range(1, L):
                    z = z + plsc.load_gather(dotbuf, [idx0 + (gbase + l)])
                gate = 1.0 / (1.0 + jnp.exp(-(z + bgv)))
                gidv = gid_buf[pl.ds(g * L, L)]
                row = [hv_buf[g * L, pl.ds(L * k, L)] for k in range(KV)]
                for j2 in range(L):
                    cur = row
                    if j2 + 1 < L:
                        row = [hv_buf[g * L + j2 + 1, pl.ds(L * k, L)]
                               for k in range(KV)]
                    gs = jnp.full((L,), gate[j2], jnp.float32)
                    base = gidv[j2] * ROWW
                    for k in range(KV):
                        plsc.addupdate(acc.at[pl.ds(base + L * k, L)],
                                       gs * cur[k])
                    plsc.addupdate(acc.at[pl.ds(base + D, L)], gs)
                return 0

            lax.fori_loop(0, C // L, group_body, 0)

    issue(0, hv_a, gid_a, sem_a)

    def pipe_body(i, _):
        ta = 2 * i
        tb = ta + 1
        wait(ta, hv_a, gid_a, sem_a)
        issue(tb, hv_b, gid_b, sem_b)
        process(ta, hv_a, gid_a)
        wait(tb, hv_b, gid_b, sem_b)

        @pl.when(tb + 1 < TPW)
        def _():
            issue(tb + 1, hv_a, gid_a, sem_a)

        process(tb, hv_b, gid_b)
        return 0

    lax.fori_loop(0, TPW // 2, pipe_body, 0)

    pltpu.sync_copy(acc, part_hbm.at[wid])


@functools.partial(
    pl.kernel,
    out_type=jax.ShapeDtypeStruct((NW, ACC_W), jnp.float32),
    mesh=plsc.VectorSubcoreMesh(core_axis_name="c", subcore_axis_name="s"),
    compiler_params=pltpu.CompilerParams(needs_layout_passes=False),
    scratch_types=[
        pltpu.VMEM((C, D), jnp.float32),
        pltpu.VMEM((C, D), jnp.float32),
        pltpu.VMEM((C + L,), jnp.int32),
        pltpu.VMEM((C + L,), jnp.int32),
        pltpu.VMEM((D,), jnp.float32),
        pltpu.VMEM((L,), jnp.float32),
        pltpu.VMEM((ACC_W,), jnp.float32),
        pltpu.VMEM((C * L,), jnp.float32),
        pltpu.SemaphoreType.DMA,
        pltpu.SemaphoreType.DMA,
    ],
)
def _sc_segment_pass(*refs):
    _sc_body(*refs)


def _combine_body(part_ref, wg_ref, bg_ref, out_ref):
    part = part_ref[...]
    s = jnp.sum(part[:, :, :D], axis=0)
    c = jnp.sum(part[:, :, D], axis=0)
    out = jnp.dot(s, wg_ref[...], preferred_element_type=jnp.float32)
    out_ref[...] = out + c[:, None] * bg_ref[...][None, :]


def kernel(hv, graph_ids, W_gate, b_gate, W_g, b_g):
    gid = graph_ids.astype(jnp.int32)
    wg_flat = W_gate.reshape(D)
    bg16 = jnp.broadcast_to(b_gate.reshape(1), (L,)).astype(jnp.float32)

    part = _sc_segment_pass(hv, gid, wg_flat, bg16)
    part3 = part.reshape(NW, NUM_GRAPHS, ROWW)

    out = pl.pallas_call(
        _combine_body,
        out_shape=jax.ShapeDtypeStruct((NUM_GRAPHS, D_GRAPH), jnp.float32),
    )(part3, W_g, b_g)
    return out
